# initial kernel scaffold (unmeasured)
import jax
import jax.numpy as jnp
from jax import lax
from jax.experimental import pallas as pl
from jax.experimental.pallas import tpu as pltpu


def kernel(x, assign, W1, W2):
    T, D = x.shape
    E, _, F = W1.shape
    T2 = 2 * T

    assign2 = assign.reshape(T, 1)

    def exch_body(x_ref, a_ref, xa_ref, aa_ref, sems):
        my_x = lax.axis_index("x")
        my_y = lax.axis_index("y")
        my_z = lax.axis_index("z")
        peer = (my_x, my_y, 1 - my_z)

        barrier = pltpu.get_barrier_semaphore()
        pl.semaphore_signal(barrier, inc=1, device_id=peer,
                            device_id_type=pl.DeviceIdType.MESH)
        pl.semaphore_wait(barrier, 1)

        xa_ref[pl.ds(0, T), :] = x_ref[...].astype(jnp.bfloat16)
        aa_ref[pl.ds(0, T), :] = a_ref[...]

        rx = pltpu.make_async_remote_copy(
            src_ref=xa_ref.at[pl.ds(0, T), :],
            dst_ref=xa_ref.at[pl.ds(T, T), :],
            send_sem=sems.at[0],
            recv_sem=sems.at[1],
            device_id=peer,
            device_id_type=pl.DeviceIdType.MESH,
        )
        ra = pltpu.make_async_remote_copy(
            src_ref=aa_ref.at[pl.ds(0, T), :],
            dst_ref=aa_ref.at[pl.ds(T, T), :],
            send_sem=sems.at[2],
            recv_sem=sems.at[3],
            device_id=peer,
            device_id_type=pl.DeviceIdType.MESH,
        )
        rx.start()
        ra.start()
        rx.wait()
        ra.wait()

    x_all, a_all = pl.pallas_call(
        exch_body,
        out_shape=(
            jax.ShapeDtypeStruct((T2, D), jnp.bfloat16),
            jax.ShapeDtypeStruct((T2, 1), jnp.int32),
        ),
        in_specs=[
            pl.BlockSpec(memory_space=pltpu.VMEM),
            pl.BlockSpec(memory_space=pltpu.VMEM),
        ],
        out_specs=(
            pl.BlockSpec(memory_space=pltpu.VMEM),
            pl.BlockSpec(memory_space=pltpu.VMEM),
        ),
        scratch_shapes=[pltpu.SemaphoreType.DMA((4,))],
        compiler_params=pltpu.CompilerParams(collective_id=0),
    )(x, assign2)

    BT = 512
    NB = T2 // BT

    def moe_body(xa_ref, aa_ref, w1_ref, w2_ref, o_ref, w1b, w2b):
        e = pl.program_id(0)
        b = pl.program_id(1)
        my_z = lax.axis_index("z")
        e_global = my_z * E + e

        @pl.when(b == 0)
        def _():
            w1b[...] = w1_ref[0].astype(jnp.bfloat16)
            w2b[...] = w2_ref[0].astype(jnp.bfloat16)

        h = jnp.dot(xa_ref[...], w1b[...], preferred_element_type=jnp.float32)
        h = jnp.maximum(h, 0.0).astype(jnp.bfloat16)
        y = jnp.dot(h, w2b[...], preferred_element_type=jnp.float32)
        mask = aa_ref[...] == e_global
        y = jnp.where(mask, y, 0.0)

        @pl.when(e == 0)
        def _():
            o_ref[pl.ds(b * BT, BT), :] = y

        @pl.when(e != 0)
        def _():
            o_ref[pl.ds(b * BT, BT), :] = o_ref[pl.ds(b * BT, BT), :] + y

    out_all = pl.pallas_call(
        moe_body,
        grid=(E, NB),
        out_shape=jax.ShapeDtypeStruct((T2, D), jnp.float32),
        in_specs=[
            pl.BlockSpec((BT, D), lambda e, b: (b, 0)),
            pl.BlockSpec((BT, 1), lambda e, b: (b, 0)),
            pl.BlockSpec((1, D, F), lambda e, b: (e, 0, 0)),
            pl.BlockSpec((1, F, D), lambda e, b: (e, 0, 0)),
        ],
        out_specs=pl.BlockSpec((T2, D), lambda e, b: (0, 0)),
        scratch_shapes=[
            pltpu.VMEM((D, F), jnp.bfloat16),
            pltpu.VMEM((F, D), jnp.bfloat16),
        ],
        compiler_params=pltpu.CompilerParams(
            dimension_semantics=("arbitrary", "arbitrary"),
        ),
    )(x_all, a_all, W1, W2)

    def comb_body(o_ref, fin_ref, sbuf, rbuf, sems):
        my_x = lax.axis_index("x")
        my_y = lax.axis_index("y")
        my_z = lax.axis_index("z")
        peer = (my_x, my_y, 1 - my_z)

        barrier = pltpu.get_barrier_semaphore()
        pl.semaphore_signal(barrier, inc=1, device_id=peer,
                            device_id_type=pl.DeviceIdType.MESH)
        pl.semaphore_wait(barrier, 1)

        sbuf[...] = o_ref[pl.ds(T, T), :].astype(jnp.bfloat16)
        r = pltpu.make_async_remote_copy(
            src_ref=sbuf,
            dst_ref=rbuf,
            send_sem=sems.at[0],
            recv_sem=sems.at[1],
            device_id=peer,
            device_id_type=pl.DeviceIdType.MESH,
        )
        r.start()
        r.wait()
        fin_ref[...] = o_ref[pl.ds(0, T), :] + rbuf[...].astype(jnp.float32)

    return pl.pallas_call(
        comb_body,
        out_shape=jax.ShapeDtypeStruct((T, D), jnp.float32),
        in_specs=[pl.BlockSpec(memory_space=pltpu.VMEM)],
        out_specs=pl.BlockSpec(memory_space=pltpu.VMEM),
        scratch_shapes=[
            pltpu.VMEM((T, D), jnp.bfloat16),
            pltpu.VMEM((T, D), jnp.bfloat16),
            pltpu.SemaphoreType.DMA((2,)),
        ],
        compiler_params=pltpu.CompilerParams(collective_id=1),
    )(out_all)


# baseline (device time: 311512 ns/iter reference)
import jax
import jax.numpy as jnp
from jax import lax
from jax.experimental import pallas as pl
from jax.experimental.pallas import tpu as pltpu


def kernel(x, assign, W1, W2):
    T, D = x.shape
    E, _, F = W1.shape
    T2 = 2 * T

    assign2 = assign.reshape(T, 1)

    def exch_body(x_ref, a_ref, xa_ref, aa_ref, sems):
        my_x = lax.axis_index("x")
        my_y = lax.axis_index("y")
        my_z = lax.axis_index("z")
        peer = (my_x, my_y, 1 - my_z)

        barrier = pltpu.get_barrier_semaphore()
        pl.semaphore_signal(barrier, inc=1, device_id=peer,
                            device_id_type=pl.DeviceIdType.MESH)
        pl.semaphore_wait(barrier, 1)

        xa_ref[pl.ds(0, T), :] = x_ref[...].astype(jnp.bfloat16)
        aa_ref[pl.ds(0, T), :] = a_ref[...]

        rx = pltpu.make_async_remote_copy(
            src_ref=xa_ref.at[pl.ds(0, T), :],
            dst_ref=xa_ref.at[pl.ds(T, T), :],
            send_sem=sems.at[0],
            recv_sem=sems.at[1],
            device_id=peer,
            device_id_type=pl.DeviceIdType.MESH,
        )
        ra = pltpu.make_async_remote_copy(
            src_ref=aa_ref.at[pl.ds(0, T), :],
            dst_ref=aa_ref.at[pl.ds(T, T), :],
            send_sem=sems.at[2],
            recv_sem=sems.at[3],
            device_id=peer,
            device_id_type=pl.DeviceIdType.MESH,
        )
        rx.start()
        ra.start()
        rx.wait()
        ra.wait()

    x_all, a_all = pl.pallas_call(
        exch_body,
        out_shape=(
            jax.ShapeDtypeStruct((T2, D), jnp.bfloat16),
            jax.ShapeDtypeStruct((T2, 1), jnp.int32),
        ),
        in_specs=[
            pl.BlockSpec(memory_space=pltpu.VMEM),
            pl.BlockSpec(memory_space=pltpu.VMEM),
        ],
        out_specs=(
            pl.BlockSpec(memory_space=pltpu.VMEM),
            pl.BlockSpec(memory_space=pltpu.VMEM),
        ),
        scratch_shapes=[pltpu.SemaphoreType.DMA((4,))],
        compiler_params=pltpu.CompilerParams(
            collective_id=0, vmem_limit_bytes=100 * 1024 * 1024
        ),
    )(x, assign2)

    BT = 512
    NB = T2 // BT

    def moe_body(xa_ref, aa_ref, w1_ref, w2_ref, o_ref, w1b, w2b):
        e = pl.program_id(0)
        b = pl.program_id(1)
        my_z = lax.axis_index("z")
        e_global = my_z * E + e

        @pl.when(b == 0)
        def _():
            w1b[...] = w1_ref[0].astype(jnp.bfloat16)
            w2b[...] = w2_ref[0].astype(jnp.bfloat16)

        h = jnp.dot(xa_ref[...], w1b[...], preferred_element_type=jnp.float32)
        h = jnp.maximum(h, 0.0).astype(jnp.bfloat16)
        y = jnp.dot(h, w2b[...], preferred_element_type=jnp.float32)
        mask = aa_ref[...] == e_global
        y = jnp.where(mask, y, 0.0)

        @pl.when(e == 0)
        def _():
            o_ref[pl.ds(b * BT, BT), :] = y

        @pl.when(e != 0)
        def _():
            o_ref[pl.ds(b * BT, BT), :] = o_ref[pl.ds(b * BT, BT), :] + y

    out_all = pl.pallas_call(
        moe_body,
        grid=(E, NB),
        out_shape=jax.ShapeDtypeStruct((T2, D), jnp.float32),
        in_specs=[
            pl.BlockSpec((BT, D), lambda e, b: (b, 0)),
            pl.BlockSpec((BT, 1), lambda e, b: (b, 0)),
            pl.BlockSpec((1, D, F), lambda e, b: (e, 0, 0)),
            pl.BlockSpec((1, F, D), lambda e, b: (e, 0, 0)),
        ],
        out_specs=pl.BlockSpec((T2, D), lambda e, b: (0, 0)),
        scratch_shapes=[
            pltpu.VMEM((D, F), jnp.bfloat16),
            pltpu.VMEM((F, D), jnp.bfloat16),
        ],
        compiler_params=pltpu.CompilerParams(
            dimension_semantics=("arbitrary", "arbitrary"),
            vmem_limit_bytes=100 * 1024 * 1024,
        ),
    )(x_all, a_all, W1, W2)

    def comb_body(o_ref, fin_ref, sbuf, rbuf, sems):
        my_x = lax.axis_index("x")
        my_y = lax.axis_index("y")
        my_z = lax.axis_index("z")
        peer = (my_x, my_y, 1 - my_z)

        barrier = pltpu.get_barrier_semaphore()
        pl.semaphore_signal(barrier, inc=1, device_id=peer,
                            device_id_type=pl.DeviceIdType.MESH)
        pl.semaphore_wait(barrier, 1)

        sbuf[...] = o_ref[pl.ds(T, T), :].astype(jnp.bfloat16)
        r = pltpu.make_async_remote_copy(
            src_ref=sbuf,
            dst_ref=rbuf,
            send_sem=sems.at[0],
            recv_sem=sems.at[1],
            device_id=peer,
            device_id_type=pl.DeviceIdType.MESH,
        )
        r.start()
        r.wait()
        fin_ref[...] = o_ref[pl.ds(0, T), :] + rbuf[...].astype(jnp.float32)

    return pl.pallas_call(
        comb_body,
        out_shape=jax.ShapeDtypeStruct((T, D), jnp.float32),
        in_specs=[pl.BlockSpec(memory_space=pltpu.VMEM)],
        out_specs=pl.BlockSpec(memory_space=pltpu.VMEM),
        scratch_shapes=[
            pltpu.VMEM((T, D), jnp.bfloat16),
            pltpu.VMEM((T, D), jnp.bfloat16),
            pltpu.SemaphoreType.DMA((2,)),
        ],
        compiler_params=pltpu.CompilerParams(
            collective_id=1, vmem_limit_bytes=100 * 1024 * 1024
        ),
    )(out_all)


# device time: 208173 ns/iter; 1.4964x vs baseline; 1.4964x over previous
import jax
import jax.numpy as jnp
from jax import lax
from jax.experimental import pallas as pl
from jax.experimental.pallas import tpu as pltpu


def kernel(x, assign, W1, W2):
    T, D = x.shape
    E, _, F = W1.shape

    perm = jnp.argsort(assign)
    inv = jnp.argsort(perm)
    xs = jnp.take(x, perm, axis=0).astype(jnp.bfloat16)
    asg = jnp.take(assign, perm).reshape(T, 1)
    W1b = W1.astype(jnp.bfloat16)
    W2b = W2.astype(jnp.bfloat16)

    NCH = 4
    CH = T // NCH
    BT = 256
    NBL = CH // BT

    def body(xs_ref, asg_ref, w1_ref, w2_ref, fin_ref,
             xrem, arem, oloc, orem, orecv, sx, rx, sa, ra, rs, rr):
        my_x = lax.axis_index("x")
        my_y = lax.axis_index("y")
        my_z = lax.axis_index("z")
        peer = (my_x, my_y, 1 - my_z)
        ebase = my_z * E

        barrier = pltpu.get_barrier_semaphore()
        pl.semaphore_signal(barrier, inc=1, device_id=peer,
                            device_id_type=pl.DeviceIdType.MESH)
        pl.semaphore_wait(barrier, 1)

        rdma_x = pltpu.make_async_remote_copy(
            src_ref=xs_ref, dst_ref=xrem, send_sem=sx, recv_sem=rx,
            device_id=peer, device_id_type=pl.DeviceIdType.MESH)
        rdma_a = pltpu.make_async_remote_copy(
            src_ref=asg_ref, dst_ref=arem, send_sem=sa, recv_sem=ra,
            device_id=peer, device_id_type=pl.DeviceIdType.MESH)
        rdma_x.start()
        rdma_a.start()

        def lblk(b, carry):
            sl = pl.ds(b * BT, BT)
            xb = xs_ref[sl, :]
            ab = asg_ref[sl, :]
            oloc[sl, :] = jnp.zeros((BT, D), jnp.bfloat16)
            for e in range(E):
                eg = ebase + e

                @pl.when(jnp.any(ab == eg))
                def _():
                    h = jnp.dot(xb, w1_ref[e],
                                preferred_element_type=jnp.float32)
                    h = jnp.maximum(h, 0.0).astype(jnp.bfloat16)
                    y = jnp.dot(h, w2_ref[e],
                                preferred_element_type=jnp.float32)
                    oloc[sl, :] = oloc[sl, :] + jnp.where(
                        ab == eg, y, 0.0).astype(jnp.bfloat16)
            return carry

        lax.fori_loop(0, T // BT, lblk, 0)

        rdma_x.wait()
        rdma_a.wait()

        sends = []
        for c in range(NCH):
            def rblk(b, carry):
                sl = pl.ds(b * BT, BT)
                xb = xrem[sl, :]
                ab = arem[sl, :]
                orem[sl, :] = jnp.zeros((BT, D), jnp.bfloat16)
                for e in range(E):
                    eg = ebase + e

                    @pl.when(jnp.any(ab == eg))
                    def _():
                        h = jnp.dot(xb, w1_ref[e],
                                    preferred_element_type=jnp.float32)
                        h = jnp.maximum(h, 0.0).astype(jnp.bfloat16)
                        y = jnp.dot(h, w2_ref[e],
                                    preferred_element_type=jnp.float32)
                        orem[sl, :] = orem[sl, :] + jnp.where(
                            ab == eg, y, 0.0).astype(jnp.bfloat16)
                return carry

            lax.fori_loop(c * NBL, (c + 1) * NBL, rblk, 0)
            r = pltpu.make_async_remote_copy(
                src_ref=orem.at[pl.ds(c * CH, CH), :],
                dst_ref=orecv.at[pl.ds(c * CH, CH), :],
                send_sem=rs.at[c], recv_sem=rr.at[c],
                device_id=peer, device_id_type=pl.DeviceIdType.MESH)
            r.start()
            sends.append(r)

        for c in range(NCH):
            sends[c].wait()
            sl = pl.ds(c * CH, CH)
            fin_ref[sl, :] = oloc[sl, :] + orecv[sl, :]

    fin = pl.pallas_call(
        body,
        out_shape=jax.ShapeDtypeStruct((T, D), jnp.bfloat16),
        in_specs=[
            pl.BlockSpec(memory_space=pltpu.VMEM),
            pl.BlockSpec(memory_space=pltpu.VMEM),
            pl.BlockSpec(memory_space=pltpu.VMEM),
            pl.BlockSpec(memory_space=pltpu.VMEM),
        ],
        out_specs=pl.BlockSpec(memory_space=pltpu.VMEM),
        scratch_shapes=[
            pltpu.VMEM((T, D), jnp.bfloat16),
            pltpu.VMEM((T, 1), jnp.int32),
            pltpu.VMEM((T, D), jnp.bfloat16),
            pltpu.VMEM((T, D), jnp.bfloat16),
            pltpu.VMEM((T, D), jnp.bfloat16),
            pltpu.SemaphoreType.DMA,
            pltpu.SemaphoreType.DMA,
            pltpu.SemaphoreType.DMA,
            pltpu.SemaphoreType.DMA,
            pltpu.SemaphoreType.DMA((NCH,)),
            pltpu.SemaphoreType.DMA((NCH,)),
        ],
        compiler_params=pltpu.CompilerParams(
            collective_id=0, vmem_limit_bytes=110 * 1024 * 1024
        ),
    )(xs, asg, W1b, W2b)

    return jnp.take(fin, inv, axis=0).astype(jnp.float32)
